# baseline (speedup 1.0000x reference)
import jax
import jax.numpy as jnp
from jax.experimental import pallas as pl

_N_CLASSES = 128
_ROWS = 16384
_COLS = 26
_BLOCK = 256


def _onehot_body(x_ref, o_ref):
    xv = x_ref[...]
    iota = jax.lax.broadcasted_iota(jnp.int32, (_BLOCK, _COLS, _N_CLASSES), 2)
    o_ref[...] = (xv[:, :, None] == iota).astype(jnp.int32)


def kernel(x):
    grid = _ROWS // _BLOCK
    return pl.pallas_call(
        _onehot_body,
        grid=(grid,),
        in_specs=[pl.BlockSpec((_BLOCK, _COLS), lambda i: (i, 0))],
        out_specs=pl.BlockSpec((_BLOCK, _COLS, _N_CLASSES), lambda i: (i, 0, 0)),
        out_shape=jax.ShapeDtypeStruct((_ROWS, _COLS, _N_CLASSES), jnp.int32),
    )(x)
